# baseline (speedup 1.0000x reference)
import functools

import jax
import jax.numpy as jnp
from jax import lax
from jax.experimental import pallas as pl
from jax.experimental.pallas import tpu as pltpu
from jax.experimental.pallas import tpu_sc as plsc

K = 16


def _tc_body(pall_ref, msf_ref, gf_ref, w0gT_ref, w0mT_ref, w1T_ref, b0_ref,
             b1_ref, zT_ref, idx_ref):
    P = pall_ref[...]
    B, C, N = msf_ref.shape
    BN = P.shape[0]

    blocks = []
    for b in range(B):
        X = msf_ref[b]
        Pb = P[b * N:(b + 1) * N]
        G = lax.dot_general(Pb, X, (((1,), (0,)), ((), ())),
                            preferred_element_type=jnp.float32)
        sqc = jnp.sum(Pb * Pb, axis=1, keepdims=True)
        sqr = jnp.sum(X * X, axis=0, keepdims=True)
        blocks.append(sqc + sqr - 2.0 * G)
    d2 = jnp.concatenate(blocks, axis=0)

    acc0 = jnp.zeros((BN, K), dtype=jnp.int32)

    def step(t, carry):
        d2c, acc = carry
        iota = lax.broadcasted_iota(jnp.int32, (BN, N), 1)
        lane_k = lax.broadcasted_iota(jnp.int32, (BN, K), 1)
        m = jnp.min(d2c, axis=1, keepdims=True)
        c = jnp.where(d2c <= m, iota, N)
        am = jnp.min(c, axis=1, keepdims=True)
        acc = jnp.where(lane_k == t, am, acc)
        d2c = jnp.where(c == am, jnp.inf, d2c)
        return d2c, acc

    d2f, acc = lax.fori_loop(0, K - 1, step, (d2, acc0))
    iota = lax.broadcasted_iota(jnp.int32, (BN, N), 1)
    lane_k = lax.broadcasted_iota(jnp.int32, (BN, K), 1)
    mF = jnp.min(d2f, axis=1, keepdims=True)
    amF = jnp.min(jnp.where(d2f <= mF, iota, N), axis=1, keepdims=True)
    acc = jnp.where(lane_k == K - 1, amF, acc)
    roff = lax.broadcasted_iota(jnp.int32, (BN, K), 0) // N * N
    idx_ref[...] = acc + roff

    g0 = lax.dot_general(gf_ref[...], w0gT_ref[...], (((1,), (0,)), ((), ())),
                         preferred_element_type=jnp.float32)
    H = g0.shape[1]
    g0b = jnp.concatenate(
        [jnp.broadcast_to(g0[b:b + 1], (N, H)) for b in range(B)], axis=0)
    h = lax.dot_general(P, w0mT_ref[...], (((1,), (0,)), ((), ())),
                        preferred_element_type=jnp.float32)
    h = jnp.maximum(h + g0b + b0_ref[...], 0.0)
    z = lax.dot_general(h, w1T_ref[...], (((1,), (0,)), ((), ())),
                        preferred_element_type=jnp.float32)
    zT_ref[...] = jnp.maximum(z + b1_ref[...], 0.0)


def _tc_call(pall, msf, gf, w0gT, w0mT, w1T, b0r, b1r):
    BN, C = pall.shape
    B = msf.shape[0]
    H = w0mT.shape[1]
    O = w1T.shape[1]
    Cg = gf.shape[1]
    full = lambda shape: pl.BlockSpec(shape, lambda: (0,) * len(shape))
    return pl.pallas_call(
        _tc_body,
        in_specs=[
            full((BN, C)),
            full((B, C, BN // B)),
            full((B, Cg)),
            full((Cg, H)),
            full((C, H)),
            full((H, O)),
            full((1, H)),
            full((1, O)),
        ],
        out_specs=[
            full((BN, O)),
            full((BN, K)),
        ],
        out_shape=[
            jax.ShapeDtypeStruct((BN, O), jnp.float32),
            jax.ShapeDtypeStruct((BN, K), jnp.int32),
        ],
    )(pall, msf, gf, w0gT, w0mT, w1T, b0r, b1r)


_NC, _NS, _L = 2, 16, 16
_NW = _NC * _NS


def _sc_gather_max(idx_flat, z_packed, D):
    PTS, DW = z_packed.shape
    PPW = PTS // _NW
    CP = 8
    NCH = PPW // CP
    ROWS = CP * K
    mesh = plsc.VectorSubcoreMesh(core_axis_name="c", subcore_axis_name="s")

    @functools.partial(
        pl.kernel, mesh=mesh,
        out_type=jax.ShapeDtypeStruct((PTS, D), jnp.float32),
        scratch_types=[
            pltpu.VMEM((PPW * K,), jnp.int32),
            pltpu.VMEM((ROWS, DW), jnp.int32),
            pltpu.VMEM((ROWS, DW), jnp.int32),
            pltpu.VMEM((CP, D), jnp.float32),
            pltpu.SemaphoreType.DMA,
            pltpu.SemaphoreType.DMA,
        ],
    )
    def body(idx_hbm, z_hbm, out_hbm, idx_v, rows0, rows1, outc_v, sem0,
             sem1):
        wid = lax.axis_index("s") * _NC + lax.axis_index("c")
        pltpu.sync_copy(idx_hbm.at[pl.ds(wid * PPW * K, PPW * K)], idx_v)
        bufs = (rows0, rows1)
        sems = (sem0, sem1)
        pltpu.async_copy(z_hbm.at[idx_v.at[pl.ds(0, ROWS)]], rows0, sem0)

        def pair(i, _):
            for par in range(2):
                c = i * 2 + par
                rows_v = bufs[par]
                @pl.when(c + 1 < NCH)
                def _():
                    pltpu.async_copy(
                        z_hbm.at[idx_v.at[pl.ds((c + 1) * ROWS, ROWS)]],
                        bufs[1 - par], sems[1 - par])

                pltpu.make_async_copy(
                    z_hbm.at[idx_v.at[pl.ds(c * ROWS, ROWS)]], rows_v,
                    sems[par]).wait()

                def point(p, _):
                    for g in range(DW // _L):
                        ws = [rows_v[p * K + r, g * _L:(g + 1) * _L]
                              for r in range(K)]
                        los = [lax.bitcast_convert_type(w << 16, jnp.float32)
                               for w in ws]
                        his = [lax.bitcast_convert_type(
                                   w & jnp.int32(-65536), jnp.float32)
                               for w in ws]
                        for vs, half in ((los, 0), (his, 1)):
                            while len(vs) > 1:
                                vs = [jnp.maximum(vs[i], vs[i + 1])
                                      for i in range(0, len(vs), 2)]
                            outc_v[p, (g * 2 + half) * _L:
                                   (g * 2 + half + 1) * _L] = vs[0]
                    return 0

                lax.fori_loop(0, CP, point, 0)
                pltpu.sync_copy(outc_v,
                                out_hbm.at[pl.ds(wid * PPW + c * CP, CP)])
            return 0

        lax.fori_loop(0, NCH // 2, pair, 0)

    return body(idx_flat, z_packed)


def kernel(global_feature, msf, W0, b0, W1, b1):
    B, C, N = msf.shape
    Cg = global_feature.shape[1]
    pall = jnp.transpose(msf, (0, 2, 1)).reshape(B * N, C)
    w0gT = jnp.transpose(W0[:, :Cg])
    w0mT = jnp.transpose(W0[:, Cg:])
    w1T = jnp.transpose(W1)
    zT, idx = _tc_call(pall, msf, global_feature, w0gT, w0mT, w1T,
                       b0[None, :], b1[None, :])
    D = zT.shape[-1]
    zb = zT.astype(jnp.bfloat16)
    zp = zb.reshape(B * N, D // 32, 2, 16).transpose(0, 1, 3, 2)
    zp = jax.lax.bitcast_convert_type(zp, jnp.int32).reshape(B * N, D // 2)
    out = _sc_gather_max(idx.reshape(-1), zp, D)
    return jnp.transpose(out.reshape(B, N, D), (0, 2, 1))


# speedup vs baseline: 1.0060x; 1.0060x over previous
import functools

import jax
import jax.numpy as jnp
from jax import lax
from jax.experimental import pallas as pl
from jax.experimental.pallas import tpu as pltpu
from jax.experimental.pallas import tpu_sc as plsc

K = 16


def _tc_body(pall_ref, msf_ref, gf_ref, w0gT_ref, w0mT_ref, w1T_ref, b0_ref,
             b1_ref, zT_ref, idx_ref):
    P = pall_ref[...]
    B, C, N = msf_ref.shape
    BN = P.shape[0]

    blocks = []
    for b in range(B):
        X = msf_ref[b]
        Pb = P[b * N:(b + 1) * N]
        G = lax.dot_general(Pb, X, (((1,), (0,)), ((), ())),
                            preferred_element_type=jnp.float32)
        sqc = jnp.sum(Pb * Pb, axis=1, keepdims=True)
        sqr = jnp.sum(X * X, axis=0, keepdims=True)
        blocks.append(sqc + sqr - 2.0 * G)
    d2 = jnp.concatenate(blocks, axis=0)

    acc0 = jnp.zeros((BN, K), dtype=jnp.int32)

    def step(t, carry):
        d2c, acc = carry
        iota = lax.broadcasted_iota(jnp.int32, (BN, N), 1)
        lane_k = lax.broadcasted_iota(jnp.int32, (BN, K), 1)
        m = jnp.min(d2c, axis=1, keepdims=True)
        am = jnp.min(jnp.where(d2c <= m, iota, N), axis=1,
                     keepdims=True)
        acc = jnp.where(lane_k == t, am, acc)
        d2c = jnp.where(iota == am, jnp.inf, d2c)
        return d2c, acc

    d2f, acc = lax.fori_loop(0, K - 1, step, (d2, acc0))
    iota = lax.broadcasted_iota(jnp.int32, (BN, N), 1)
    lane_k = lax.broadcasted_iota(jnp.int32, (BN, K), 1)
    mF = jnp.min(d2f, axis=1, keepdims=True)
    amF = jnp.min(jnp.where(d2f <= mF, iota, N), axis=1, keepdims=True)
    acc = jnp.where(lane_k == K - 1, amF, acc)
    roff = lax.broadcasted_iota(jnp.int32, (BN, K), 0) // N * N
    idx_ref[...] = acc + roff

    g0 = lax.dot_general(gf_ref[...], w0gT_ref[...], (((1,), (0,)), ((), ())),
                         preferred_element_type=jnp.float32)
    H = g0.shape[1]
    g0b = jnp.concatenate(
        [jnp.broadcast_to(g0[b:b + 1], (N, H)) for b in range(B)], axis=0)
    h = lax.dot_general(P, w0mT_ref[...], (((1,), (0,)), ((), ())),
                        preferred_element_type=jnp.float32)
    h = jnp.maximum(h + g0b + b0_ref[...], 0.0)
    z = lax.dot_general(h, w1T_ref[...], (((1,), (0,)), ((), ())),
                        preferred_element_type=jnp.float32)
    zT_ref[...] = jnp.maximum(z + b1_ref[...], 0.0)


def _tc_call(pall, msf, gf, w0gT, w0mT, w1T, b0r, b1r):
    BN, C = pall.shape
    B = msf.shape[0]
    H = w0mT.shape[1]
    O = w1T.shape[1]
    Cg = gf.shape[1]
    full = lambda shape: pl.BlockSpec(shape, lambda: (0,) * len(shape))
    return pl.pallas_call(
        _tc_body,
        in_specs=[
            full((BN, C)),
            full((B, C, BN // B)),
            full((B, Cg)),
            full((Cg, H)),
            full((C, H)),
            full((H, O)),
            full((1, H)),
            full((1, O)),
        ],
        out_specs=[
            full((BN, O)),
            full((BN, K)),
        ],
        out_shape=[
            jax.ShapeDtypeStruct((BN, O), jnp.float32),
            jax.ShapeDtypeStruct((BN, K), jnp.int32),
        ],
    )(pall, msf, gf, w0gT, w0mT, w1T, b0r, b1r)


_NC, _NS, _L = 2, 16, 16
_NW = _NC * _NS


def _sc_gather_max(idx_flat, z_packed, D):
    PTS, DW = z_packed.shape
    PPW = PTS // _NW
    CP = 8
    NCH = PPW // CP
    ROWS = CP * K
    mesh = plsc.VectorSubcoreMesh(core_axis_name="c", subcore_axis_name="s")

    @functools.partial(
        pl.kernel, mesh=mesh,
        out_type=jax.ShapeDtypeStruct((PTS, D), jnp.float32),
        scratch_types=[
            pltpu.VMEM((PPW * K,), jnp.int32),
            pltpu.VMEM((ROWS, DW), jnp.int32),
            pltpu.VMEM((ROWS, DW), jnp.int32),
            pltpu.VMEM((CP, D), jnp.float32),
            pltpu.SemaphoreType.DMA,
            pltpu.SemaphoreType.DMA,
        ],
    )
    def body(idx_hbm, z_hbm, out_hbm, idx_v, rows0, rows1, outc_v, sem0,
             sem1):
        wid = lax.axis_index("s") * _NC + lax.axis_index("c")
        pltpu.sync_copy(idx_hbm.at[pl.ds(wid * PPW * K, PPW * K)], idx_v)
        bufs = (rows0, rows1)
        sems = (sem0, sem1)
        pltpu.async_copy(z_hbm.at[idx_v.at[pl.ds(0, ROWS)]], rows0, sem0)

        def pair(i, _):
            for par in range(2):
                c = i * 2 + par
                rows_v = bufs[par]
                @pl.when(c + 1 < NCH)
                def _():
                    pltpu.async_copy(
                        z_hbm.at[idx_v.at[pl.ds((c + 1) * ROWS, ROWS)]],
                        bufs[1 - par], sems[1 - par])

                pltpu.make_async_copy(
                    z_hbm.at[idx_v.at[pl.ds(c * ROWS, ROWS)]], rows_v,
                    sems[par]).wait()

                def point(p, _):
                    for g in range(DW // _L):
                        ws = [rows_v[p * K + r, g * _L:(g + 1) * _L]
                              for r in range(K)]
                        los = [lax.bitcast_convert_type(w << 16, jnp.float32)
                               for w in ws]
                        his = [lax.bitcast_convert_type(
                                   w & jnp.int32(-65536), jnp.float32)
                               for w in ws]
                        for vs, half in ((los, 0), (his, 1)):
                            while len(vs) > 1:
                                vs = [jnp.maximum(vs[i], vs[i + 1])
                                      for i in range(0, len(vs), 2)]
                            outc_v[p, (g * 2 + half) * _L:
                                   (g * 2 + half + 1) * _L] = vs[0]
                    return 0

                lax.fori_loop(0, CP, point, 0)
                pltpu.sync_copy(outc_v,
                                out_hbm.at[pl.ds(wid * PPW + c * CP, CP)])
            return 0

        lax.fori_loop(0, NCH // 2, pair, 0)

    return body(idx_flat, z_packed)


def kernel(global_feature, msf, W0, b0, W1, b1):
    B, C, N = msf.shape
    Cg = global_feature.shape[1]
    pall = jnp.transpose(msf, (0, 2, 1)).reshape(B * N, C)
    w0gT = jnp.transpose(W0[:, :Cg])
    w0mT = jnp.transpose(W0[:, Cg:])
    w1T = jnp.transpose(W1)
    zT, idx = _tc_call(pall, msf, global_feature, w0gT, w0mT, w1T,
                       b0[None, :], b1[None, :])
    D = zT.shape[-1]
    zb = zT.astype(jnp.bfloat16)
    zp = zb.reshape(B * N, D // 32, 2, 16).transpose(0, 1, 3, 2)
    zp = jax.lax.bitcast_convert_type(zp, jnp.int32).reshape(B * N, D // 2)
    out = _sc_gather_max(idx.reshape(-1), zp, D)
    return jnp.transpose(out.reshape(B, N, D), (0, 2, 1))
